# baseline (speedup 1.0000x reference)
import functools

import jax
import jax.numpy as jnp
from jax.experimental import pallas as pl
from jax.experimental.pallas import tpu as pltpu

B, S, D = 32, 4096, 768
K = S // 4
BS = 2048

_I32_MIN = -2147483648
_I32_TOPMASK = 2147483647


def _stage1_body(x_ref, w_ref, b_ref, score_ref, lp0_ref, lp1_ref, ent_ref):
    xb = x_ref[0]
    logits = jnp.dot(xb, w_ref[...], preferred_element_type=jnp.float32)
    logits = logits + b_ref[...]
    l0 = logits[:, 0:1]
    l1 = logits[:, 1:2]
    m = jnp.maximum(l0, l1)
    lse = m + jnp.log(jnp.exp(l0 - m) + jnp.exp(l1 - m))
    lp0 = l0 - lse
    lp1 = l1 - lse
    ent = -(jnp.exp(lp0) * lp0 + jnp.exp(lp1) * lp1)
    score = l1 - l0
    score_ref[...] = score.reshape(1, 1, 1, BS)
    lp0_ref[...] = lp0.reshape(1, 1, 1, BS)
    lp1_ref[...] = lp1.reshape(1, 1, 1, BS)
    ent_ref[...] = ent.reshape(1, 1, 1, BS)


def _sortable_key(score):
    bits = jax.lax.bitcast_convert_type(score, jnp.int32)
    return jnp.where(bits < 0, bits ^ jnp.int32(_I32_TOPMASK), bits)


def _stage2_body(score_ref, lp0_ref, lp1_ref, act_ref, lp_ref):
    key = _sortable_key(score_ref[...])

    def bit_step(i, t_u):
        cand = t_u | jnp.left_shift(jnp.int32(1), 31 - i)
        scand = cand ^ jnp.int32(_I32_MIN)
        cnt = jnp.sum((key >= scand).astype(jnp.int32), axis=1, keepdims=True)
        return jnp.where(cnt >= K, cand, t_u)

    t_u = jax.lax.fori_loop(0, 32, bit_step, jnp.zeros((B, 1), jnp.int32))
    thr = t_u ^ jnp.int32(_I32_MIN)

    gt = key > thr
    eq = key == thr
    c_gt = jnp.sum(gt.astype(jnp.int32), axis=1, keepdims=True)
    need = K - c_gt

    idx = jax.lax.broadcasted_iota(jnp.int32, (B, S), 1)

    def idx_step(i, t_j):
        cand = t_j | jnp.left_shift(jnp.int32(1), 11 - i)
        cnt = jnp.sum((eq & (idx <= cand)).astype(jnp.int32), axis=1,
                      keepdims=True)
        return jnp.where(cnt <= need, cand, t_j)

    t_j = jax.lax.fori_loop(0, 12, idx_step, jnp.zeros((B, 1), jnp.int32))

    mask = gt | (eq & (idx <= t_j))
    act_ref[...] = mask.astype(jnp.int32)
    lp_ref[...] = jnp.where(mask, lp1_ref[...], lp0_ref[...])


@jax.jit
def kernel(x, W, b):
    b2 = b.reshape(1, 2)
    fdef = jax.ShapeDtypeStruct((B, S), jnp.float32)
    f4def = jax.ShapeDtypeStruct((B, S // BS, 1, BS), jnp.float32)
    ospec = pl.BlockSpec((1, 1, 1, BS), lambda i, j: (i, j, 0, 0))
    score, lp0, lp1, ent = pl.pallas_call(
        _stage1_body,
        grid=(B, S // BS),
        in_specs=[
            pl.BlockSpec((1, BS, D), lambda i, j: (i, j, 0)),
            pl.BlockSpec((D, 2), lambda i, j: (0, 0)),
            pl.BlockSpec((1, 2), lambda i, j: (0, 0)),
        ],
        out_specs=[ospec, ospec, ospec, ospec],
        out_shape=[f4def, f4def, f4def, f4def],
        compiler_params=pltpu.CompilerParams(
            dimension_semantics=("parallel", "arbitrary")),
    )(x, W, b2)
    score = score.reshape(B, S)
    lp0 = lp0.reshape(B, S)
    lp1 = lp1.reshape(B, S)
    ent = ent.reshape(B, S)

    actions, log_probs = pl.pallas_call(
        _stage2_body,
        out_shape=[jax.ShapeDtypeStruct((B, S), jnp.int32), fdef],
    )(score, lp0, lp1)

    topk_log_probs = jnp.zeros((B, S), jnp.float32)
    return (actions, topk_log_probs, log_probs, ent)


# speedup vs baseline: 1.1278x; 1.1278x over previous
import jax
import jax.numpy as jnp
from jax.experimental import pallas as pl
from jax.experimental.pallas import tpu as pltpu

B, S, D = 32, 4096, 768
K = S // 4
BS = 2048

_I32_MIN = -2147483648
_I32_TOPMASK = 2147483647


def _stage1_body(x_ref, w_ref, out_ref):
    out_ref[0] = jnp.dot(x_ref[0], w_ref[...],
                         preferred_element_type=jnp.float32)


def _sortable_key(score):
    bits = jax.lax.bitcast_convert_type(score, jnp.int32)
    return jnp.where(bits < 0, bits ^ jnp.int32(_I32_TOPMASK), bits)


def _stage2_body(l0_ref, l1_ref, b_ref, act_ref, lp_ref, ent_ref):
    l0 = l0_ref[...] + b_ref[0, 0]
    l1 = l1_ref[...] + b_ref[0, 1]
    score = l1 - l0
    m = jnp.maximum(l0, l1)
    lse = m + jnp.log(jnp.exp(l0 - m) + jnp.exp(l1 - m))
    lp0 = l0 - lse
    lp1 = l1 - lse
    ent_ref[...] = -(jnp.exp(lp0) * lp0 + jnp.exp(lp1) * lp1)

    key = _sortable_key(score)

    def bit_step(i, t_u):
        cand = t_u | jnp.left_shift(jnp.int32(1), 31 - i)
        scand = cand ^ jnp.int32(_I32_MIN)
        cnt = jnp.sum((key >= scand).astype(jnp.int32), axis=1, keepdims=True)
        return jnp.where(cnt >= K, cand, t_u)

    t_u = jax.lax.fori_loop(0, 32, bit_step, jnp.zeros((B, 1), jnp.int32))
    thr = t_u ^ jnp.int32(_I32_MIN)

    gt = key > thr
    eq = key == thr
    c_gt = jnp.sum(gt.astype(jnp.int32), axis=1, keepdims=True)
    need = K - c_gt

    idx = jax.lax.broadcasted_iota(jnp.int32, (B, S), 1)

    def idx_step(i, t_j):
        cand = t_j | jnp.left_shift(jnp.int32(1), 11 - i)
        cnt = jnp.sum((eq & (idx <= cand)).astype(jnp.int32), axis=1,
                      keepdims=True)
        return jnp.where(cnt <= need, cand, t_j)

    t_j = jax.lax.fori_loop(0, 12, idx_step, jnp.zeros((B, 1), jnp.int32))

    mask = gt | (eq & (idx <= t_j))
    act_ref[...] = mask.astype(jnp.int32)
    lp_ref[...] = jnp.where(mask, lp1, lp0)


@jax.jit
def kernel(x, W, b):
    nblk = B * S // BS
    x3 = x.reshape(nblk, BS, D)
    logits = pl.pallas_call(
        _stage1_body,
        grid=(nblk,),
        in_specs=[
            pl.BlockSpec((1, BS, D), lambda i: (i, 0, 0)),
            pl.BlockSpec((D, 2), lambda i: (0, 0)),
        ],
        out_specs=pl.BlockSpec((1, BS, 2), lambda i: (i, 0, 0)),
        out_shape=jax.ShapeDtypeStruct((nblk, BS, 2), jnp.float32),
        compiler_params=pltpu.CompilerParams(
            dimension_semantics=("arbitrary",)),
    )(x3, W)

    lr = logits.reshape(B, S, 2)
    l0 = lr[:, :, 0]
    l1 = lr[:, :, 1]

    fdef = jax.ShapeDtypeStruct((B, S), jnp.float32)
    actions, log_probs, ent = pl.pallas_call(
        _stage2_body,
        out_shape=[jax.ShapeDtypeStruct((B, S), jnp.int32), fdef, fdef],
    )(l0, l1, b.reshape(1, 2))

    topk_log_probs = jnp.zeros((B, S), jnp.float32)
    return (actions, topk_log_probs, log_probs, ent)


# speedup vs baseline: 1.4895x; 1.3207x over previous
import jax
import jax.numpy as jnp
from jax.experimental import pallas as pl
from jax.experimental.pallas import tpu as pltpu

B, S, D = 32, 4096, 768
K = S // 4
BS = 2048
GS = BS // 128

_I32_MIN = -2147483648
_I32_TOPMASK = 2147483647


def _stage1_body(x_ref, w_ref, b_ref, score_ref, lp0_ref, lp1_ref, ent_ref):
    logits = jnp.dot(x_ref[0], w_ref[...],
                     preferred_element_type=jnp.float32)
    l0 = logits[:, 0:1].reshape(GS, 128) + b_ref[0, 0]
    l1 = logits[:, 1:2].reshape(GS, 128) + b_ref[0, 1]
    m = jnp.maximum(l0, l1)
    lse = m + jnp.log(jnp.exp(l0 - m) + jnp.exp(l1 - m))
    lp0 = l0 - lse
    lp1 = l1 - lse
    score_ref[0] = l1 - l0
    lp0_ref[0] = lp0
    lp1_ref[0] = lp1
    ent_ref[0] = -(jnp.exp(lp0) * lp0 + jnp.exp(lp1) * lp1)


def _sortable_key(score):
    bits = jax.lax.bitcast_convert_type(score, jnp.int32)
    return jnp.where(bits < 0, bits ^ jnp.int32(_I32_TOPMASK), bits)


def _stage2_body(score_ref, lp0_ref, lp1_ref, act_ref, lp_ref):
    key = _sortable_key(score_ref[...])

    def bit_step(i, t_u):
        cand = t_u | jnp.left_shift(jnp.int32(1), 31 - i)
        scand = cand ^ jnp.int32(_I32_MIN)
        cnt = jnp.sum((key >= scand).astype(jnp.int32), axis=1, keepdims=True)
        return jnp.where(cnt >= K, cand, t_u)

    t_u = jax.lax.fori_loop(0, 32, bit_step, jnp.zeros((B, 1), jnp.int32))
    thr = t_u ^ jnp.int32(_I32_MIN)

    gt = key > thr
    eq = key == thr
    c_gt = jnp.sum(gt.astype(jnp.int32), axis=1, keepdims=True)
    need = K - c_gt

    idx = jax.lax.broadcasted_iota(jnp.int32, (B, S), 1)

    def idx_step(i, t_j):
        cand = t_j | jnp.left_shift(jnp.int32(1), 11 - i)
        cnt = jnp.sum((eq & (idx <= cand)).astype(jnp.int32), axis=1,
                      keepdims=True)
        return jnp.where(cnt <= need, cand, t_j)

    t_j = jax.lax.fori_loop(0, 12, idx_step, jnp.zeros((B, 1), jnp.int32))

    mask = gt | (eq & (idx <= t_j))
    act_ref[...] = mask.astype(jnp.int32)
    lp_ref[...] = jnp.where(mask, lp1_ref[...], lp0_ref[...])


@jax.jit
def kernel(x, W, b):
    nblk = B * S // BS
    x3 = x.reshape(nblk, BS, D)
    tdef = jax.ShapeDtypeStruct((nblk, GS, 128), jnp.float32)
    ospec = pl.BlockSpec((1, GS, 128), lambda i: (i, 0, 0))
    score, lp0, lp1, ent = pl.pallas_call(
        _stage1_body,
        grid=(nblk,),
        in_specs=[
            pl.BlockSpec((1, BS, D), lambda i: (i, 0, 0)),
            pl.BlockSpec((D, 2), lambda i: (0, 0)),
            pl.BlockSpec((1, 2), lambda i: (0, 0)),
        ],
        out_specs=[ospec, ospec, ospec, ospec],
        out_shape=[tdef, tdef, tdef, tdef],
        compiler_params=pltpu.CompilerParams(
            dimension_semantics=("arbitrary",)),
    )(x3, W, b.reshape(1, 2))

    score = score.reshape(B, S)
    lp0 = lp0.reshape(B, S)
    lp1 = lp1.reshape(B, S)
    ent = ent.reshape(B, S)

    fdef = jax.ShapeDtypeStruct((B, S), jnp.float32)
    actions, log_probs = pl.pallas_call(
        _stage2_body,
        out_shape=[jax.ShapeDtypeStruct((B, S), jnp.int32), fdef],
    )(score, lp0, lp1)

    topk_log_probs = jnp.zeros((B, S), jnp.float32)
    return (actions, topk_log_probs, log_probs, ent)


# speedup vs baseline: 2.5351x; 1.7019x over previous
import jax
import jax.numpy as jnp
from jax.experimental import pallas as pl
from jax.experimental.pallas import tpu as pltpu

B, S, D = 32, 4096, 768
K = S // 4
BS = 2048
NBLK = B * S // BS
RPB = S // BS

_I32_MIN = -2147483648
_I32_TOPMASK = 2147483647


def _stage1_body(x_ref, wt_ref, q_ref):
    y = jax.lax.dot_general(wt_ref[...], x_ref[0],
                            (((1,), (1,)), ((), ())),
                            preferred_element_type=jnp.float32)
    q_ref[0] = jnp.concatenate([y, jnp.zeros((6, BS), jnp.float32)], axis=0)


def _sortable_key(score):
    bits = jax.lax.bitcast_convert_type(score, jnp.int32)
    return jnp.where(bits < 0, bits ^ jnp.int32(_I32_TOPMASK), bits)


def _stage2_body(q_ref, b_ref, act_ref, lp_ref, ent_ref):
    l0 = (q_ref[:, 0, :] + b_ref[0, 0]).reshape(B, S)
    l1 = (q_ref[:, 1, :] + b_ref[0, 1]).reshape(B, S)
    score = l1 - l0
    m = jnp.maximum(l0, l1)
    lse = m + jnp.log(jnp.exp(l0 - m) + jnp.exp(l1 - m))
    lp0 = l0 - lse
    lp1 = l1 - lse
    ent_ref[...] = -(jnp.exp(lp0) * lp0 + jnp.exp(lp1) * lp1)

    key = _sortable_key(score)

    def bit_step(i, t_u):
        cand = t_u | jnp.left_shift(jnp.int32(1), 31 - i)
        scand = cand ^ jnp.int32(_I32_MIN)
        cnt = jnp.sum((key >= scand).astype(jnp.int32), axis=1, keepdims=True)
        return jnp.where(cnt >= K, cand, t_u)

    t_u = jax.lax.fori_loop(0, 32, bit_step, jnp.zeros((B, 1), jnp.int32))
    thr = t_u ^ jnp.int32(_I32_MIN)

    gt = key > thr
    eq = key == thr
    c_gt = jnp.sum(gt.astype(jnp.int32), axis=1, keepdims=True)
    need = K - c_gt

    idx = jax.lax.broadcasted_iota(jnp.int32, (B, S), 1)

    def idx_step(i, t_j):
        cand = t_j | jnp.left_shift(jnp.int32(1), 11 - i)
        cnt = jnp.sum((eq & (idx <= cand)).astype(jnp.int32), axis=1,
                      keepdims=True)
        return jnp.where(cnt <= need, cand, t_j)

    t_j = jax.lax.fori_loop(0, 12, idx_step, jnp.zeros((B, 1), jnp.int32))

    mask = gt | (eq & (idx <= t_j))
    act_ref[...] = mask.astype(jnp.int32)
    lp_ref[...] = jnp.where(mask, lp1, lp0)


@jax.jit
def kernel(x, W, b):
    x3 = x.reshape(NBLK, BS, D)
    q = pl.pallas_call(
        _stage1_body,
        grid=(NBLK,),
        in_specs=[
            pl.BlockSpec((1, BS, D), lambda i: (i, 0, 0)),
            pl.BlockSpec((2, D), lambda i: (0, 0)),
        ],
        out_specs=pl.BlockSpec((1, 8, BS), lambda i: (i, 0, 0)),
        out_shape=jax.ShapeDtypeStruct((NBLK, 8, BS), jnp.float32),
        compiler_params=pltpu.CompilerParams(
            dimension_semantics=("arbitrary",)),
    )(x3, W.T)

    fdef = jax.ShapeDtypeStruct((B, S), jnp.float32)
    actions, log_probs, ent = pl.pallas_call(
        _stage2_body,
        out_shape=[jax.ShapeDtypeStruct((B, S), jnp.int32), fdef, fdef],
    )(q, b.reshape(1, 2))

    topk_log_probs = jnp.zeros((B, S), jnp.float32)
    return (actions, topk_log_probs, log_probs, ent)
